# baseline (speedup 1.0000x reference)
import functools

import jax
import jax.numpy as jnp
from jax import lax
from jax.experimental import pallas as pl
from jax.experimental.pallas import tpu as pltpu
from jax.experimental.pallas import tpu_sc as plsc

N = 10000
D = 128
E = 320000
LAYERS = 3
EPS = 1e-12

NC = 2
NS = 16
NW = NC * NS
LANES = 16

B = 80
CH = 125
PHASES = 5
PCH = CH // PHASES
NSLOT = 4
STRIPE = 624
STRIPE_REM = N - STRIPE * NS
N_PAD = N + LANES
TRASH = N

_mesh = plsc.VectorSubcoreMesh(
    core_axis_name="c", subcore_axis_name="s", num_cores=NC, num_subcores=NS
)


@functools.partial(
    pl.kernel,
    out_type=jax.ShapeDtypeStruct((NC, N, D), jnp.float32),
    mesh=_mesh,
    scratch_types=[
        pltpu.VMEM((PCH, B), jnp.int32),
        pltpu.VMEM((PCH, B), jnp.int32),
        pltpu.VMEM((NSLOT, B, D), jnp.float32),
        pltpu.VMEM_SHARED((N_PAD, D), jnp.float32),
        pltpu.SemaphoreType.DMA,
        pltpu.SemaphoreType.DMA,
        pltpu.SemaphoreType.DMA,
        pltpu.SemaphoreType.DMA,
        pltpu.SemaphoreType.DMA,
    ],
)
def _sc_agg(xn_hbm, src_hbm, dst_hbm, out_hbm, src_v, dst_v, rows_v, acc_sh,
            gsem0, gsem1, ssem0, ssem1, isem):
    gsem = (gsem0, gsem1)
    ssem = (ssem0, ssem1)
    c = lax.axis_index("c")
    s = lax.axis_index("s")
    wid = s * NC + c
    r0 = s * STRIPE

    @pl.when(c == 0)
    def _init_xn():
        pltpu.sync_copy(xn_hbm.at[pl.ds(r0, STRIPE)],
                        acc_sh.at[pl.ds(r0, STRIPE)])

        @pl.when(s == NS - 1)
        def _init_rem():
            pltpu.sync_copy(xn_hbm.at[pl.ds(STRIPE * NS, STRIPE_REM)],
                            acc_sh.at[pl.ds(STRIPE * NS, STRIPE_REM)])

    @pl.when(c == 1)
    def _init_zero():
        zb = rows_v.at[0]

        def _z(r, carry):
            row = zb.at[r]
            for k in range(D // LANES):
                row[pl.ds(k * LANES, LANES)] = jnp.zeros((LANES,), jnp.float32)
            return carry

        lax.fori_loop(0, B, _z, 0)
        for t in range(STRIPE // B):
            pltpu.sync_copy(zb, acc_sh.at[pl.ds(r0 + t * B, B)])
        rem = STRIPE - (STRIPE // B) * B
        pltpu.sync_copy(zb.at[pl.ds(0, rem)],
                        acc_sh.at[pl.ds(r0 + STRIPE - rem, rem)])

        @pl.when(s == NS - 1)
        def _zero_rem():
            pltpu.sync_copy(zb.at[pl.ds(0, STRIPE_REM)],
                            acc_sh.at[pl.ds(STRIPE * NS, STRIPE_REM)])

    plsc.subcore_barrier()

    def _fix_chunk(sv_ref, dv_ref, j):
        row_s = sv_ref.at[j]
        row_d = dv_ref.at[j]
        for k in range(B // LANES):
            sl = pl.ds(k * LANES, LANES)
            row_d[sl] = jnp.where(row_s[sl] == row_d[sl], TRASH, row_d[sl])

    for ph in range(PHASES):
        sv = src_v
        dv = dst_v
        pltpu.async_copy(src_hbm.at[wid, ph], sv, isem)
        pltpu.async_copy(dst_hbm.at[wid, ph], dv, isem)
        pltpu.make_async_copy(src_hbm.at[wid, ph], sv, isem).wait()
        pltpu.make_async_copy(dst_hbm.at[wid, ph], dv, isem).wait()

        def _step(j, b):
            pltpu.make_async_copy(
                xn_hbm.at[sv.at[j]], rows_v.at[b], gsem[b % 2]).wait()

            def _wait_prev():
                pltpu.make_async_copy(
                    rows_v.at[(b + 2) % NSLOT],
                    acc_sh.at[dv.at[j - 2]], ssem[b % 2]).wait()

            if isinstance(j, int):
                if j >= 2:
                    _wait_prev()
            else:
                pl.when(j >= 2)(_wait_prev)

            pltpu.async_copy(
                rows_v.at[b], acc_sh.at[dv.at[j]], ssem[b % 2], add=True)

            def _issue_next():
                pltpu.async_copy(
                    xn_hbm.at[sv.at[j + 2]],
                    rows_v.at[(b + 2) % NSLOT], gsem[b % 2])
                _fix_chunk(sv, dv, j + 2)

            if isinstance(j, int):
                if j + 2 < PCH:
                    _issue_next()
            else:
                pl.when(j + 2 < PCH)(_issue_next)

        _fix_chunk(sv, dv, 0)
        _fix_chunk(sv, dv, 1)
        pltpu.async_copy(xn_hbm.at[sv.at[0]], rows_v.at[0], gsem[0])
        pltpu.async_copy(xn_hbm.at[sv.at[1]], rows_v.at[1], gsem[1])

        def _grp(g, carry):
            for b in range(NSLOT):
                _step(g * NSLOT + b, b)
            return carry

        lax.fori_loop(0, PCH // NSLOT, _grp, 0)

        for j in range((PCH // NSLOT) * NSLOT, PCH):
            _step(j, j % NSLOT)
        for j in (PCH - 2, PCH - 1):
            pltpu.make_async_copy(
                rows_v.at[j % NSLOT], acc_sh.at[dv.at[j]],
                ssem[j % 2]).wait()

    plsc.subcore_barrier()

    pltpu.sync_copy(acc_sh.at[pl.ds(r0, STRIPE)], out_hbm.at[c, pl.ds(r0, STRIPE)])

    @pl.when(s == NS - 1)
    def _out_rem():
        pltpu.sync_copy(acc_sh.at[pl.ds(STRIPE * NS, STRIPE_REM)],
                        out_hbm.at[c, pl.ds(STRIPE * NS, STRIPE_REM)])


RB = 400


def _dot(a, b):
    return lax.dot_general(a, b, (((1,), (0,)), ((), ())),
                           precision=lax.Precision.HIGHEST)


def _pre_body(x_ref, wr_ref, b_ref, xn_ref, r_ref):
    x = x_ref[...]
    n = jnp.sqrt(jnp.sum(x * x, axis=-1, keepdims=True))
    xn_ref[...] = x / jnp.maximum(n, EPS)
    r_ref[...] = _dot(x, wr_ref[...]) + b_ref[...]


_pre = pl.pallas_call(
    _pre_body,
    grid=(N // RB,),
    in_specs=[
        pl.BlockSpec((RB, D), lambda i: (i, 0)),
        pl.BlockSpec((D, D), lambda i: (0, 0)),
        pl.BlockSpec((1, D), lambda i: (0, 0)),
    ],
    out_specs=[pl.BlockSpec((RB, D), lambda i: (i, 0)),
               pl.BlockSpec((RB, D), lambda i: (i, 0))],
    out_shape=[jax.ShapeDtypeStruct((N, D), jnp.float32),
               jax.ShapeDtypeStruct((N, D), jnp.float32)],
)


def _layer_core(p_ref, h_ref, r_ref, wl_ref, relu):
    h = h_ref[...]
    agg = p_ref[0] + p_ref[1]
    an = jnp.sqrt(jnp.sum(agg * agg, axis=-1, keepdims=True))
    msg = agg / jnp.maximum(an, EPS)
    hnorm = jnp.sqrt(jnp.sum(h * h, axis=-1, keepdims=True))
    out = _dot(msg, wl_ref[...]) * hnorm + r_ref[...]
    on = jnp.sqrt(jnp.sum(out * out, axis=-1, keepdims=True))
    out = out / jnp.maximum(on, EPS)
    if relu:
        out = jnp.maximum(out, 0.0)
    return h + out


def _mid_body(p_ref, h_ref, r_ref, wl_ref, wrn_ref, bn_ref,
              h_out_ref, xn_out_ref, r_out_ref):
    hn = _layer_core(p_ref, h_ref, r_ref, wl_ref, relu=True)
    h_out_ref[...] = hn
    nn = jnp.sqrt(jnp.sum(hn * hn, axis=-1, keepdims=True))
    xn_out_ref[...] = hn / jnp.maximum(nn, EPS)
    r_out_ref[...] = _dot(hn, wrn_ref[...]) + bn_ref[...]


_layer_mid = pl.pallas_call(
    _mid_body,
    grid=(N // RB,),
    in_specs=[
        pl.BlockSpec((NC, RB, D), lambda i: (0, i, 0)),
        pl.BlockSpec((RB, D), lambda i: (i, 0)),
        pl.BlockSpec((RB, D), lambda i: (i, 0)),
        pl.BlockSpec((D, D), lambda i: (0, 0)),
        pl.BlockSpec((D, D), lambda i: (0, 0)),
        pl.BlockSpec((1, D), lambda i: (0, 0)),
    ],
    out_specs=[pl.BlockSpec((RB, D), lambda i: (i, 0)),
               pl.BlockSpec((RB, D), lambda i: (i, 0)),
               pl.BlockSpec((RB, D), lambda i: (i, 0))],
    out_shape=[jax.ShapeDtypeStruct((N, D), jnp.float32),
               jax.ShapeDtypeStruct((N, D), jnp.float32),
               jax.ShapeDtypeStruct((N, D), jnp.float32)],
)


def _last_body(p_ref, h_ref, r_ref, wl_ref, h_out_ref):
    h_out_ref[...] = _layer_core(p_ref, h_ref, r_ref, wl_ref, relu=False)


_layer_last = pl.pallas_call(
    _last_body,
    grid=(N // RB,),
    in_specs=[
        pl.BlockSpec((NC, RB, D), lambda i: (0, i, 0)),
        pl.BlockSpec((RB, D), lambda i: (i, 0)),
        pl.BlockSpec((RB, D), lambda i: (i, 0)),
        pl.BlockSpec((D, D), lambda i: (0, 0)),
    ],
    out_specs=pl.BlockSpec((RB, D), lambda i: (i, 0)),
    out_shape=jax.ShapeDtypeStruct((N, D), jnp.float32),
)


def kernel(x, edge_index, W_l, b_l, W_r, scale):
    src_p = edge_index[0].reshape(NW, PHASES, PCH, B)
    dst_p = edge_index[1].reshape(NW, PHASES, PCH, B)

    h = x
    xn, r = _pre(x, W_r[0], b_l[0].reshape(1, D))
    for i in range(LAYERS):
        p = _sc_agg(xn, src_p, dst_p)
        wl_s = W_l[i] * scale[i]
        if i < LAYERS - 1:
            h, xn, r = _layer_mid(p, h, r, wl_s, W_r[i + 1],
                                  b_l[i + 1].reshape(1, D))
        else:
            h = _layer_last(p, h, r, wl_s)
    return h


# speedup vs baseline: 1.0355x; 1.0355x over previous
import functools

import jax
import jax.numpy as jnp
from jax import lax
from jax.experimental import pallas as pl
from jax.experimental.pallas import tpu as pltpu
from jax.experimental.pallas import tpu_sc as plsc

N = 10000
D = 128
E = 320000
LAYERS = 3
EPS = 1e-12

NC = 2
NS = 16
NW = NC * NS
LANES = 16

B = 80
CH = 125
PHASES = 5
PCH = CH // PHASES
NSLOT = 4
STRIPE = 624
STRIPE_REM = N - STRIPE * NS
N_PAD = N + LANES
TRASH = N

_mesh = plsc.VectorSubcoreMesh(
    core_axis_name="c", subcore_axis_name="s", num_cores=NC, num_subcores=NS
)


@functools.partial(
    pl.kernel,
    out_type=jax.ShapeDtypeStruct((NC, N, D), jnp.float32),
    mesh=_mesh,
    scratch_types=[
        pltpu.VMEM((PCH, B), jnp.int32),
        pltpu.VMEM((PCH, B), jnp.int32),
        pltpu.VMEM((NSLOT, B, D), jnp.float32),
        pltpu.VMEM_SHARED((N_PAD, D), jnp.float32),
        pltpu.SemaphoreType.DMA,
        pltpu.SemaphoreType.DMA,
        pltpu.SemaphoreType.DMA,
        pltpu.SemaphoreType.DMA,
        pltpu.SemaphoreType.DMA,
    ],
)
def _sc_agg(xn_hbm, src_hbm, dst_hbm, out_hbm, src_v, dst_v, rows_v, acc_sh,
            gsem0, gsem1, ssem0, ssem1, isem):
    gsem = (gsem0, gsem1)
    ssem = (ssem0, ssem1)
    c = lax.axis_index("c")
    s = lax.axis_index("s")
    wid = s * NC + c
    r0 = s * STRIPE

    @pl.when(c == 0)
    def _init_xn():
        pltpu.sync_copy(xn_hbm.at[pl.ds(r0, STRIPE)],
                        acc_sh.at[pl.ds(r0, STRIPE)])

        @pl.when(s == NS - 1)
        def _init_rem():
            pltpu.sync_copy(xn_hbm.at[pl.ds(STRIPE * NS, STRIPE_REM)],
                            acc_sh.at[pl.ds(STRIPE * NS, STRIPE_REM)])

    @pl.when(c == 1)
    def _init_zero():
        zb = rows_v.at[0]

        def _z(r, carry):
            row = zb.at[r]
            for k in range(D // LANES):
                row[pl.ds(k * LANES, LANES)] = jnp.zeros((LANES,), jnp.float32)
            return carry

        lax.fori_loop(0, B, _z, 0)
        for t in range(STRIPE // B):
            pltpu.sync_copy(zb, acc_sh.at[pl.ds(r0 + t * B, B)])
        rem = STRIPE - (STRIPE // B) * B
        pltpu.sync_copy(zb.at[pl.ds(0, rem)],
                        acc_sh.at[pl.ds(r0 + STRIPE - rem, rem)])

        @pl.when(s == NS - 1)
        def _zero_rem():
            pltpu.sync_copy(zb.at[pl.ds(0, STRIPE_REM)],
                            acc_sh.at[pl.ds(STRIPE * NS, STRIPE_REM)])

    plsc.subcore_barrier()

    def _fix_chunk(sv_ref, dv_ref, j):
        row_s = sv_ref.at[j]
        row_d = dv_ref.at[j]
        for k in range(B // LANES):
            sl = pl.ds(k * LANES, LANES)
            row_d[sl] = jnp.where(row_s[sl] == row_d[sl], TRASH, row_d[sl])

    for ph in range(PHASES):
        sv = src_v
        dv = dst_v
        pltpu.async_copy(src_hbm.at[wid, ph], sv, isem)
        pltpu.async_copy(dst_hbm.at[wid, ph], dv, isem)
        pltpu.make_async_copy(src_hbm.at[wid, ph], sv, isem).wait()
        pltpu.make_async_copy(dst_hbm.at[wid, ph], dv, isem).wait()

        def _step(j, b):
            pltpu.make_async_copy(
                xn_hbm.at[sv.at[j]], rows_v.at[b], gsem[b % 2]).wait()

            def _wait_prev():
                pltpu.make_async_copy(
                    rows_v.at[(b + 2) % NSLOT],
                    acc_sh.at[dv.at[j - 2]], ssem[b % 2]).wait()

            if isinstance(j, int):
                if j >= 2:
                    _wait_prev()
            else:
                pl.when(j >= 2)(_wait_prev)

            pltpu.async_copy(
                rows_v.at[b], acc_sh.at[dv.at[j]], ssem[b % 2], add=True)

            def _issue_next():
                pltpu.async_copy(
                    xn_hbm.at[sv.at[j + 2]],
                    rows_v.at[(b + 2) % NSLOT], gsem[b % 2])
                _fix_chunk(sv, dv, j + 2)

            if isinstance(j, int):
                if j + 2 < PCH:
                    _issue_next()
            else:
                pl.when(j + 2 < PCH)(_issue_next)

        _fix_chunk(sv, dv, 0)
        _fix_chunk(sv, dv, 1)
        pltpu.async_copy(xn_hbm.at[sv.at[0]], rows_v.at[0], gsem[0])
        pltpu.async_copy(xn_hbm.at[sv.at[1]], rows_v.at[1], gsem[1])

        def _grp(g, carry):
            for b in range(NSLOT):
                _step(g * NSLOT + b, b)
            return carry

        lax.fori_loop(0, PCH // NSLOT, _grp, 0)

        for j in range((PCH // NSLOT) * NSLOT, PCH):
            _step(j, j % NSLOT)
        for j in (PCH - 2, PCH - 1):
            pltpu.make_async_copy(
                rows_v.at[j % NSLOT], acc_sh.at[dv.at[j]],
                ssem[j % 2]).wait()

    plsc.subcore_barrier()

    pltpu.sync_copy(acc_sh.at[pl.ds(r0, STRIPE)], out_hbm.at[c, pl.ds(r0, STRIPE)])

    @pl.when(s == NS - 1)
    def _out_rem():
        pltpu.sync_copy(acc_sh.at[pl.ds(STRIPE * NS, STRIPE_REM)],
                        out_hbm.at[c, pl.ds(STRIPE * NS, STRIPE_REM)])


RB = 400


def _dot(a, b):
    return lax.dot_general(a, b, (((1,), (0,)), ((), ())),
                           precision=lax.Precision.HIGHEST)


def _prenorm_body(x_ref, xn_ref):
    x = x_ref[...]
    n = jnp.sqrt(jnp.sum(x * x, axis=-1, keepdims=True))
    xn_ref[...] = x / jnp.maximum(n, EPS)


_prenorm = pl.pallas_call(
    _prenorm_body,
    grid=(N // RB,),
    in_specs=[pl.BlockSpec((RB, D), lambda i: (i, 0))],
    out_specs=pl.BlockSpec((RB, D), lambda i: (i, 0)),
    out_shape=jax.ShapeDtypeStruct((N, D), jnp.float32),
)


def _root_body(h_ref, wr_ref, b_ref, r_ref):
    r_ref[...] = _dot(h_ref[...], wr_ref[...]) + b_ref[...]


_root = pl.pallas_call(
    _root_body,
    grid=(N // RB,),
    in_specs=[
        pl.BlockSpec((RB, D), lambda i: (i, 0)),
        pl.BlockSpec((D, D), lambda i: (0, 0)),
        pl.BlockSpec((1, D), lambda i: (0, 0)),
    ],
    out_specs=pl.BlockSpec((RB, D), lambda i: (i, 0)),
    out_shape=jax.ShapeDtypeStruct((N, D), jnp.float32),
)


def _layer_core(p_ref, h_ref, r_ref, wl_ref, relu):
    h = h_ref[...]
    agg = p_ref[0] + p_ref[1]
    an = jnp.sqrt(jnp.sum(agg * agg, axis=-1, keepdims=True))
    msg = agg / jnp.maximum(an, EPS)
    hnorm = jnp.sqrt(jnp.sum(h * h, axis=-1, keepdims=True))
    out = _dot(msg, wl_ref[...]) * hnorm + r_ref[...]
    on = jnp.sqrt(jnp.sum(out * out, axis=-1, keepdims=True))
    out = out / jnp.maximum(on, EPS)
    if relu:
        out = jnp.maximum(out, 0.0)
    return h + out


def _mid_body(p_ref, h_ref, r_ref, wl_ref, h_out_ref, xn_out_ref):
    hn = _layer_core(p_ref, h_ref, r_ref, wl_ref, relu=True)
    h_out_ref[...] = hn
    nn = jnp.sqrt(jnp.sum(hn * hn, axis=-1, keepdims=True))
    xn_out_ref[...] = hn / jnp.maximum(nn, EPS)


_layer_mid = pl.pallas_call(
    _mid_body,
    grid=(N // RB,),
    in_specs=[
        pl.BlockSpec((NC, RB, D), lambda i: (0, i, 0)),
        pl.BlockSpec((RB, D), lambda i: (i, 0)),
        pl.BlockSpec((RB, D), lambda i: (i, 0)),
        pl.BlockSpec((D, D), lambda i: (0, 0)),
    ],
    out_specs=[pl.BlockSpec((RB, D), lambda i: (i, 0)),
               pl.BlockSpec((RB, D), lambda i: (i, 0))],
    out_shape=[jax.ShapeDtypeStruct((N, D), jnp.float32),
               jax.ShapeDtypeStruct((N, D), jnp.float32)],
)


def _last_body(p_ref, h_ref, r_ref, wl_ref, h_out_ref):
    h_out_ref[...] = _layer_core(p_ref, h_ref, r_ref, wl_ref, relu=False)


_layer_last = pl.pallas_call(
    _last_body,
    grid=(N // RB,),
    in_specs=[
        pl.BlockSpec((NC, RB, D), lambda i: (0, i, 0)),
        pl.BlockSpec((RB, D), lambda i: (i, 0)),
        pl.BlockSpec((RB, D), lambda i: (i, 0)),
        pl.BlockSpec((D, D), lambda i: (0, 0)),
    ],
    out_specs=pl.BlockSpec((RB, D), lambda i: (i, 0)),
    out_shape=jax.ShapeDtypeStruct((N, D), jnp.float32),
)


def kernel(x, edge_index, W_l, b_l, W_r, scale):
    src_p = edge_index[0].reshape(NW, PHASES, PCH, B)
    dst_p = edge_index[1].reshape(NW, PHASES, PCH, B)

    h = x
    xn = _prenorm(x)
    for i in range(LAYERS):
        p = _sc_agg(xn, src_p, dst_p)
        wl_s = W_l[i] * scale[i]
        r = _root(h, W_r[i], b_l[i].reshape(1, D))
        if i < LAYERS - 1:
            h, xn = _layer_mid(p, h, r, wl_s)
        else:
            h = _layer_last(p, h, r, wl_s)
    return h
